# baseline (speedup 1.0000x reference)
import jax
import jax.numpy as jnp
from jax import lax
from jax.experimental import pallas as pl
from jax.experimental.pallas import tpu as pltpu
from jax.experimental.pallas import tpu_sc as plsc

D_MODEL = 128
WINDOW = 128
NBUF = 5
LAG = 2
N_UNITS = 32

TC_COUNT = 102400
TC_BLK = 512


def _sc_gather(table, idx_sc):
    n = idx_sc.shape[0]
    per_unit = n // N_UNITS
    nsteps = per_unit // WINDOW
    rounds = nsteps // NBUF

    mesh = plsc.VectorSubcoreMesh(core_axis_name="core",
                                  subcore_axis_name="subcore")

    @pl.kernel(out_type=jax.ShapeDtypeStruct((n, D_MODEL), table.dtype),
               mesh=mesh,
               scratch_types=[pltpu.VMEM((per_unit,), jnp.int32),
                              pltpu.VMEM((NBUF, WINDOW, D_MODEL),
                                         jnp.float32),
                              pltpu.SemaphoreType.DMA((NBUF,)),
                              pltpu.SemaphoreType.DMA((NBUF,)),
                              pltpu.SemaphoreType.DMA])
    def gather_kernel(table_hbm, idx_hbm, out_hbm, idx_v, rows_v, gsem,
                      wsem, isem):
        wid = lax.axis_index("subcore") * 2 + lax.axis_index("core")
        unit_base = wid * per_unit

        pltpu.async_copy(idx_hbm.at[pl.ds(unit_base, per_unit)], idx_v,
                         isem).wait()

        def gather(slot, step):
            return pltpu.make_async_copy(
                table_hbm.at[idx_v.at[pl.ds(step * WINDOW, WINDOW)]],
                rows_v.at[slot], gsem.at[slot])

        def wb(slot, step):
            return pltpu.make_async_copy(
                rows_v.at[slot],
                out_hbm.at[pl.ds(unit_base + step * WINDOW, WINDOW)],
                wsem.at[slot])

        for t in range(LAG):
            gather(t % NBUF, t).start()
        for t in range(LAG, NBUF):
            gather(t % NBUF, t).start()
            gather((t - LAG) % NBUF, t - LAG).wait()
            wb((t - LAG) % NBUF, t - LAG).start()

        @pl.loop(1, rounds)
        def _(r):
            for slot in range(NBUF):
                t = r * NBUF + slot
                wb(slot, t - NBUF).wait()
                gather(slot, t).start()
                s2 = (slot + NBUF - LAG) % NBUF
                gather(s2, t - LAG).wait()
                wb(s2, t - LAG).start()

        for step in range(nsteps, nsteps + LAG):
            gather((step - LAG) % NBUF, step - LAG).wait()
            wb((step - LAG) % NBUF, step - LAG).start()
        for step in range(nsteps - NBUF + LAG, nsteps):
            wb(step % NBUF, step).wait()

    return gather_kernel(table, idx_sc)


def _tc_gather(table, idx_tc):
    n = idx_tc.shape[0]
    vocab = table.shape[0]

    def body(idx_ref, table_ref, out_ref):
        j = pl.program_id(0)
        base = j * TC_BLK

        def row(i, _):
            v = idx_ref[base + i]
            out_ref[pl.ds(i, 1), :] = table_ref[pl.ds(v, 1), :]
            return 0

        lax.fori_loop(0, TC_BLK, row, 0)

    grid_spec = pltpu.PrefetchScalarGridSpec(
        num_scalar_prefetch=1,
        grid=(n // TC_BLK,),
        in_specs=[pl.BlockSpec((vocab, D_MODEL), lambda j, idx: (0, 0))],
        out_specs=pl.BlockSpec((TC_BLK, D_MODEL), lambda j, idx: (j, 0)),
    )
    return pl.pallas_call(
        body,
        grid_spec=grid_spec,
        out_shape=jax.ShapeDtypeStruct((n, D_MODEL), table.dtype),
        compiler_params=pltpu.CompilerParams(
            dimension_semantics=("arbitrary",)),
    )(idx_tc, table)


def kernel(x, table):
    b, s = x.shape
    n = b * s
    idx = x.reshape(n).astype(jnp.int32)

    out_tc = _tc_gather(table, idx[:TC_COUNT])
    out_sc = _sc_gather(table, idx[TC_COUNT:])
    out = jnp.concatenate([out_tc, out_sc], axis=0)
    return out.reshape(b, s, D_MODEL)


# speedup vs baseline: 3.0912x; 3.0912x over previous
import jax
import jax.numpy as jnp
from jax import lax
from jax.experimental import pallas as pl
from jax.experimental.pallas import tpu as pltpu
from jax.experimental.pallas import tpu_sc as plsc

D_MODEL = 128
WINDOW = 128
NBUF = 5
LAG = 2
N_UNITS = 32


def kernel(x, table):
    b, s = x.shape
    n = b * s
    per_unit = n // N_UNITS
    nsteps = per_unit // WINDOW
    rounds = nsteps // NBUF
    idx = x.reshape(n).astype(jnp.int32)

    mesh = plsc.VectorSubcoreMesh(core_axis_name="core",
                                  subcore_axis_name="subcore")

    @pl.kernel(out_type=jax.ShapeDtypeStruct((n, D_MODEL), table.dtype),
               mesh=mesh,
               scratch_types=[pltpu.VMEM((per_unit,), jnp.int32),
                              pltpu.VMEM((NBUF, WINDOW, D_MODEL),
                                         jnp.float32),
                              pltpu.SemaphoreType.DMA((NBUF,)),
                              pltpu.SemaphoreType.DMA((NBUF,)),
                              pltpu.SemaphoreType.DMA])
    def gather_kernel(table_hbm, idx_hbm, out_hbm, idx_v, rows_v, gsem,
                      wsem, isem):
        wid = lax.axis_index("subcore") * 2 + lax.axis_index("core")
        unit_base = wid * per_unit

        pltpu.async_copy(idx_hbm.at[pl.ds(unit_base, per_unit)], idx_v,
                         isem).wait()

        def gather(slot, step):
            return pltpu.make_async_copy(
                table_hbm.at[idx_v.at[pl.ds(step * WINDOW, WINDOW)]],
                rows_v.at[slot], gsem.at[slot])

        def wb(slot, step):
            return pltpu.make_async_copy(
                rows_v.at[slot],
                out_hbm.at[pl.ds(unit_base + step * WINDOW, WINDOW)],
                wsem.at[slot])

        for t in range(LAG):
            gather(t % NBUF, t).start()
        for t in range(LAG, NBUF):
            gather(t % NBUF, t).start()
            gather((t - LAG) % NBUF, t - LAG).wait()
            wb((t - LAG) % NBUF, t - LAG).start()

        @pl.loop(1, rounds)
        def _(r):
            for slot in range(NBUF):
                t = r * NBUF + slot
                wb(slot, t - NBUF).wait()
                gather(slot, t).start()
                s2 = (slot + NBUF - LAG) % NBUF
                gather(s2, t - LAG).wait()
                wb(s2, t - LAG).start()

        for step in range(nsteps, nsteps + LAG):
            gather((step - LAG) % NBUF, step - LAG).wait()
            wb((step - LAG) % NBUF, step - LAG).start()
        for step in range(nsteps - NBUF + LAG, nsteps):
            wb(step % NBUF, step).wait()

    out = gather_kernel(table, idx)
    return out.reshape(b, s, D_MODEL)


# speedup vs baseline: 3.1087x; 1.0057x over previous
import jax
import jax.numpy as jnp
from jax import lax
from jax.experimental import pallas as pl
from jax.experimental.pallas import tpu as pltpu
from jax.experimental.pallas import tpu_sc as plsc

D_MODEL = 128
WINDOW = 128
NBUF = 5
LAG = 3
N_UNITS = 32


def kernel(x, table):
    b, s = x.shape
    n = b * s
    per_unit = n // N_UNITS
    nsteps = per_unit // WINDOW
    rounds = nsteps // NBUF
    idx = x.reshape(n).astype(jnp.int32)

    mesh = plsc.VectorSubcoreMesh(core_axis_name="core",
                                  subcore_axis_name="subcore")

    @pl.kernel(out_type=jax.ShapeDtypeStruct((n, D_MODEL), table.dtype),
               mesh=mesh,
               scratch_types=[pltpu.VMEM((per_unit,), jnp.int32),
                              pltpu.VMEM((NBUF, WINDOW, D_MODEL),
                                         jnp.float32),
                              pltpu.SemaphoreType.DMA((NBUF,)),
                              pltpu.SemaphoreType.DMA((NBUF,)),
                              pltpu.SemaphoreType.DMA])
    def gather_kernel(table_hbm, idx_hbm, out_hbm, idx_v, rows_v, gsem,
                      wsem, isem):
        wid = lax.axis_index("subcore") * 2 + lax.axis_index("core")
        unit_base = wid * per_unit

        pltpu.async_copy(idx_hbm.at[pl.ds(unit_base, per_unit)], idx_v,
                         isem).wait()

        def gather(slot, step):
            return pltpu.make_async_copy(
                table_hbm.at[idx_v.at[pl.ds(step * WINDOW, WINDOW)]],
                rows_v.at[slot], gsem.at[slot])

        def wb(slot, step):
            return pltpu.make_async_copy(
                rows_v.at[slot],
                out_hbm.at[pl.ds(unit_base + step * WINDOW, WINDOW)],
                wsem.at[slot])

        for t in range(LAG):
            gather(t % NBUF, t).start()
        for t in range(LAG, NBUF):
            gather(t % NBUF, t).start()
            gather((t - LAG) % NBUF, t - LAG).wait()
            wb((t - LAG) % NBUF, t - LAG).start()

        @pl.loop(1, rounds)
        def _(r):
            for slot in range(NBUF):
                t = r * NBUF + slot
                wb(slot, t - NBUF).wait()
                gather(slot, t).start()
                s2 = (slot + NBUF - LAG) % NBUF
                gather(s2, t - LAG).wait()
                wb(s2, t - LAG).start()

        for step in range(nsteps, nsteps + LAG):
            gather((step - LAG) % NBUF, step - LAG).wait()
            wb((step - LAG) % NBUF, step - LAG).start()
        for step in range(nsteps - NBUF + LAG, nsteps):
            wb(step % NBUF, step).wait()

    out = gather_kernel(table, idx)
    return out.reshape(b, s, D_MODEL)
